# baseline (speedup 1.0000x reference)
import functools

import jax
import jax.numpy as jnp
from jax import lax
from jax.experimental import pallas as pl
from jax.experimental.pallas import tpu as pltpu
from jax.experimental.pallas import tpu_sc as plsc

IN_DIM = 1024
HID = 4096
OUT_DIM = 1024
E = 8
TOP_K = 2
B = 4096

A = B * TOP_K
T = 256
NT = A // T + E
S = NT * T
NW = 32
AW = A // NW
DSUB = 8
DCH = AW // DSUB
BW = B // NW
CCH = 32
CSUB = BW // CCH
L = 16

BT = 512


def _router_kernel(x_ref, wg_ref, ltri_ref, probs_ref, i1_ref, i2_ref,
                   g1_ref, g2_ref, r1_ref, r2_ref, cnt_ref, carry):
    blk = pl.program_id(0)

    @pl.when(blk == 0)
    def _():
        carry[...] = jnp.zeros_like(carry)

    logits = jnp.dot(x_ref[...], wg_ref[...], preferred_element_type=jnp.float32)
    m = jnp.max(logits, axis=-1, keepdims=True)
    ex = jnp.exp(logits - m)
    p = ex / jnp.sum(ex, axis=-1, keepdims=True)
    probs_ref[...] = p
    iota = jax.lax.broadcasted_iota(jnp.int32, p.shape, 1)
    m1 = jnp.max(p, axis=-1, keepdims=True)
    i1 = jnp.min(jnp.where(p == m1, iota, E), axis=-1, keepdims=True)
    p2 = jnp.where(iota == i1, -1.0, p)
    m2 = jnp.max(p2, axis=-1, keepdims=True)
    i2 = jnp.min(jnp.where(p2 == m2, iota, E), axis=-1, keepdims=True)
    i1_ref[...] = i1
    i2_ref[...] = i2
    denom = m1 + m2
    g1_ref[...] = m1 / denom
    g2_ref[...] = m2 / denom

    oh1 = (iota == i1).astype(jnp.float32)
    oh2 = (iota == i2).astype(jnp.float32)
    oh = oh1 + oh2
    pre = jnp.dot(ltri_ref[...], oh, preferred_element_type=jnp.float32)
    base = carry[...] + pre
    r1 = jnp.sum(base * oh1, axis=-1, keepdims=True)
    r2 = jnp.sum(base * oh2, axis=-1, keepdims=True)
    r1_ref[...] = r1.astype(jnp.int32)
    r2_ref[...] = r2.astype(jnp.int32)
    carry[...] += jnp.sum(oh, axis=0, keepdims=True)
    cnt_ref[...] = carry[...]


def _router(x, Wg):
    return pl.pallas_call(
        _router_kernel,
        grid=(B // BT,),
        in_specs=[
            pl.BlockSpec((BT, IN_DIM), lambda i: (i, 0)),
            pl.BlockSpec((IN_DIM, E), lambda i: (0, 0)),
            pl.BlockSpec((BT, BT), lambda i: (0, 0)),
        ],
        out_specs=[
            pl.BlockSpec((BT, E), lambda i: (i, 0)),
            pl.BlockSpec((BT, 1), lambda i: (i, 0)),
            pl.BlockSpec((BT, 1), lambda i: (i, 0)),
            pl.BlockSpec((BT, 1), lambda i: (i, 0)),
            pl.BlockSpec((BT, 1), lambda i: (i, 0)),
            pl.BlockSpec((BT, 1), lambda i: (i, 0)),
            pl.BlockSpec((BT, 1), lambda i: (i, 0)),
            pl.BlockSpec((1, E), lambda i: (0, 0)),
        ],
        out_shape=[
            jax.ShapeDtypeStruct((B, E), jnp.float32),
            jax.ShapeDtypeStruct((B, 1), jnp.int32),
            jax.ShapeDtypeStruct((B, 1), jnp.int32),
            jax.ShapeDtypeStruct((B, 1), jnp.float32),
            jax.ShapeDtypeStruct((B, 1), jnp.float32),
            jax.ShapeDtypeStruct((B, 1), jnp.int32),
            jax.ShapeDtypeStruct((B, 1), jnp.int32),
            jax.ShapeDtypeStruct((1, E), jnp.float32),
        ],
        scratch_shapes=[pltpu.VMEM((1, E), jnp.float32)],
    )(x, Wg, jnp.tril(jnp.ones((BT, BT), jnp.float32), -1))


_SC_MESH = dict(core_axis_name="c", subcore_axis_name="s", num_cores=2,
                num_subcores=16)


def _worker_id():
    return lax.axis_index("s") * 2 + lax.axis_index("c")


@functools.cache
def _sc_dispatch_kernel():
    @functools.partial(
        pl.kernel,
        out_type=jax.ShapeDtypeStruct((S, IN_DIM), jnp.float32),
        mesh=plsc.VectorSubcoreMesh(**_SC_MESH),
        scratch_types=[
            pltpu.VMEM((DSUB, DCH), jnp.int32),
            pltpu.VMEM((DSUB, DCH), jnp.int32),
            pltpu.VMEM((DCH, IN_DIM), jnp.float32),
            pltpu.VMEM((DCH, IN_DIM), jnp.float32),
            pltpu.SemaphoreType.DMA,
            pltpu.SemaphoreType.DMA,
            pltpu.SemaphoreType.DMA,
            pltpu.SemaphoreType.DMA,
        ],
    )
    def body(x_hbm, tok_hbm, slot_hbm, xs_hbm, tok_v, slot_v, r0, r1,
             sg0, sg1, ss0, ss1):
        wid = _worker_id()
        pltpu.sync_copy(tok_hbm.at[wid], tok_v)
        pltpu.sync_copy(slot_hbm.at[wid], slot_v)
        bufs, sgs, sss = [r0, r1], [sg0, sg1], [ss0, ss1]
        g = pltpu.async_copy(x_hbm.at[tok_v.at[0]], r0, sg0)
        s_prev = None
        for c in range(DSUB):
            g.wait()
            if s_prev is not None:
                s_prev.wait()
            s_prev = pltpu.async_copy(
                bufs[c % 2], xs_hbm.at[slot_v.at[c]], sss[c % 2])
            if c + 1 < DSUB:
                g = pltpu.async_copy(
                    x_hbm.at[tok_v.at[c + 1]], bufs[(c + 1) % 2],
                    sgs[(c + 1) % 2])
        s_prev.wait()

    return body


def _sc_dispatch(x, tok3, slot3):
    return _sc_dispatch_kernel()(x, tok3, slot3)


@functools.cache
def _sc_combine_kernel():
    @functools.partial(
        pl.kernel,
        out_type=(
            jax.ShapeDtypeStruct((B, OUT_DIM), jnp.float32),
            jax.ShapeDtypeStruct((B, OUT_DIM), jnp.float32),
        ),
        mesh=plsc.VectorSubcoreMesh(**_SC_MESH),
        scratch_types=[
            pltpu.VMEM((CSUB, CCH), jnp.int32),
            pltpu.VMEM((CSUB, CCH), jnp.int32),
            pltpu.VMEM((CCH, OUT_DIM), jnp.float32),
            pltpu.VMEM((CCH, OUT_DIM), jnp.float32),
            pltpu.SemaphoreType.DMA,
            pltpu.SemaphoreType.DMA,
            pltpu.SemaphoreType.DMA,
            pltpu.SemaphoreType.DMA,
        ],
    )
    def body(ys_hbm, p0_hbm, p1_hbm, y0_hbm, y1_hbm, p0_v, p1_v, r0, r1,
             sg0, sg1, sw0, sw1):
        wid = _worker_id()
        base = wid * BW
        pltpu.sync_copy(p0_hbm.at[wid], p0_v)
        pltpu.sync_copy(p1_hbm.at[wid], p1_v)
        bufs, sgs, sws = [r0, r1], [sg0, sg1], [sw0, sw1]
        items = []
        for c in range(CSUB):
            items.append((p0_v, c, y0_hbm))
            items.append((p1_v, c, y1_hbm))
        pv0, c0, _ = items[0]
        g = pltpu.async_copy(ys_hbm.at[pv0.at[c0]], r0, sg0)
        w_prev = None
        for k, (pv, c, dst) in enumerate(items):
            g.wait()
            if w_prev is not None:
                w_prev.wait()
            w_prev = pltpu.async_copy(
                bufs[k % 2], dst.at[pl.ds(base + c * CCH, CCH)], sws[k % 2])
            if k + 1 < len(items):
                pvn, cn, _ = items[k + 1]
                g = pltpu.async_copy(
                    ys_hbm.at[pvn.at[cn]], bufs[(k + 1) % 2],
                    sgs[(k + 1) % 2])
        w_prev.wait()

    return body


def _sc_combine(ys, pos0, pos1):
    return _sc_combine_kernel()(ys, pos0, pos1)


def _ffn_kernel(te_ref, act_ref, xs_ref, w1_ref, b1_ref, w2_ref,
                b2_ref, ys_ref):
    g = pl.program_id(0)

    @pl.when(act_ref[g] != 0)
    def _():
        xb = xs_ref[...].astype(jnp.bfloat16)
        h = jnp.dot(xb, w1_ref[0], preferred_element_type=jnp.float32) + b1_ref[0]
        h = jnp.maximum(h, 0.0).astype(jnp.bfloat16)
        y = jnp.dot(h, w2_ref[0], preferred_element_type=jnp.float32) + b2_ref[0]
        ys_ref[...] = y


def _ffn(te, act, xs, W1b, b1r, W2b, b2r):
    grid_spec = pltpu.PrefetchScalarGridSpec(
        num_scalar_prefetch=2,
        grid=(NT,),
        in_specs=[
            pl.BlockSpec((T, IN_DIM), lambda g, te, act: (g, 0)),
            pl.BlockSpec((1, IN_DIM, HID), lambda g, te, act: (te[g], 0, 0)),
            pl.BlockSpec((1, 1, HID), lambda g, te, act: (te[g], 0, 0)),
            pl.BlockSpec((1, HID, OUT_DIM), lambda g, te, act: (te[g], 0, 0)),
            pl.BlockSpec((1, 1, OUT_DIM), lambda g, te, act: (te[g], 0, 0)),
        ],
        out_specs=pl.BlockSpec((T, OUT_DIM), lambda g, te, act: (g, 0)),
    )
    return pl.pallas_call(
        _ffn_kernel,
        grid_spec=grid_spec,
        out_shape=jax.ShapeDtypeStruct((S, OUT_DIM), jnp.float32),
    )(te, act, xs, W1b, b1r, W2b, b2r)


def _mix_kernel(y0_ref, y1_ref, g1_ref, g2_ref, out_ref):
    out_ref[...] = y0_ref[...] * g1_ref[...] + y1_ref[...] * g2_ref[...]


def _mix(y0, y1, g1, g2):
    return pl.pallas_call(
        _mix_kernel,
        grid=(B // BT,),
        in_specs=[
            pl.BlockSpec((BT, OUT_DIM), lambda i: (i, 0)),
            pl.BlockSpec((BT, OUT_DIM), lambda i: (i, 0)),
            pl.BlockSpec((BT, 1), lambda i: (i, 0)),
            pl.BlockSpec((BT, 1), lambda i: (i, 0)),
        ],
        out_specs=pl.BlockSpec((BT, OUT_DIM), lambda i: (i, 0)),
        out_shape=jax.ShapeDtypeStruct((B, OUT_DIM), jnp.float32),
    )(y0, y1, g1, g2)


NCB = 8


def _cast_kernel(w1_ref, w2_ref, o1_ref, o2_ref):
    o1_ref[...] = w1_ref[...].astype(jnp.bfloat16)
    o2_ref[...] = w2_ref[...].astype(jnp.bfloat16)


def _cast(W1, W2):
    return pl.pallas_call(
        _cast_kernel,
        grid=(E, NCB),
        in_specs=[
            pl.BlockSpec((1, IN_DIM // NCB, HID), lambda e, i: (e, i, 0)),
            pl.BlockSpec((1, HID // NCB, OUT_DIM), lambda e, i: (e, i, 0)),
        ],
        out_specs=[
            pl.BlockSpec((1, IN_DIM // NCB, HID), lambda e, i: (e, i, 0)),
            pl.BlockSpec((1, HID // NCB, OUT_DIM), lambda e, i: (e, i, 0)),
        ],
        out_shape=[
            jax.ShapeDtypeStruct((E, IN_DIM, HID), jnp.bfloat16),
            jax.ShapeDtypeStruct((E, HID, OUT_DIM), jnp.bfloat16),
        ],
    )(W1, W2)


def kernel(x, Wg, W1, b1, W2, b2):
    probs, i1, i2, g1, g2, r1, r2, cntf = _router(x, Wg)

    counts = cntf.reshape(E).astype(jnp.int32)
    ntile = (counts + T - 1) // T
    pb = jnp.concatenate([jnp.zeros(1, ntile.dtype), jnp.cumsum(ntile)[:-1]])
    er = jnp.arange(E, dtype=jnp.int32)
    pb1 = jnp.sum((i1 == er[None, :]) * pb[None, :], axis=1, keepdims=True)
    pb2 = jnp.sum((i2 == er[None, :]) * pb[None, :], axis=1, keepdims=True)
    slot1 = T * pb1 + r1
    slot2 = T * pb2 + r2
    slot = jnp.concatenate([slot1, slot2], axis=1).reshape(A)
    total_tiles = pb[E - 1] + ntile[E - 1]
    tile_expert = (
        jnp.sum(jnp.arange(NT)[:, None] >= pb[None, :], axis=1) - 1
    ).astype(jnp.int32)
    act = (jnp.arange(NT) < total_tiles).astype(jnp.int32)

    pos0 = slot1.reshape(NW, CSUB, CCH)
    pos1 = slot2.reshape(NW, CSUB, CCH)
    slot3 = slot.reshape(NW, DSUB, DCH)
    tok3 = (jnp.arange(A, dtype=jnp.int32) // TOP_K).reshape(NW, DSUB, DCH)
    xs = _sc_dispatch(x, tok3, slot3)

    W1b, W2b = _cast(W1, W2)
    ys = _ffn(
        tile_expert,
        act,
        xs,
        W1b,
        b1.reshape(E, 1, HID),
        W2b,
        b2.reshape(E, 1, OUT_DIM),
    )

    y0, y1 = _sc_combine(ys, pos0, pos1)
    out = _mix(y0, y1, g1, g2)
    return (out, probs)


# speedup vs baseline: 1.0071x; 1.0071x over previous
import functools

import jax
import jax.numpy as jnp
from jax import lax
from jax.experimental import pallas as pl
from jax.experimental.pallas import tpu as pltpu
from jax.experimental.pallas import tpu_sc as plsc

IN_DIM = 1024
HID = 4096
OUT_DIM = 1024
E = 8
TOP_K = 2
B = 4096

A = B * TOP_K
T = 256
NT = A // T + E
S = NT * T
NW = 32
AW = A // NW
DSUB = 4
DCH = AW // DSUB
BW = B // NW
CCH = 64
CSUB = BW // CCH

BT = 512


def _router_kernel(x_ref, wg_ref, ltri_ref, probs_ref, i1_ref, i2_ref,
                   g1_ref, g2_ref, r1_ref, r2_ref, cnt_ref, carry):
    blk = pl.program_id(0)

    @pl.when(blk == 0)
    def _():
        carry[...] = jnp.zeros_like(carry)

    logits = jnp.dot(x_ref[...], wg_ref[...], preferred_element_type=jnp.float32)
    m = jnp.max(logits, axis=-1, keepdims=True)
    ex = jnp.exp(logits - m)
    p = ex / jnp.sum(ex, axis=-1, keepdims=True)
    probs_ref[...] = p
    iota = jax.lax.broadcasted_iota(jnp.int32, p.shape, 1)
    m1 = jnp.max(p, axis=-1, keepdims=True)
    i1 = jnp.min(jnp.where(p == m1, iota, E), axis=-1, keepdims=True)
    p2 = jnp.where(iota == i1, -1.0, p)
    m2 = jnp.max(p2, axis=-1, keepdims=True)
    i2 = jnp.min(jnp.where(p2 == m2, iota, E), axis=-1, keepdims=True)
    i1_ref[...] = i1
    i2_ref[...] = i2
    denom = m1 + m2
    g1_ref[...] = m1 / denom
    g2_ref[...] = m2 / denom

    oh1 = (iota == i1).astype(jnp.float32)
    oh2 = (iota == i2).astype(jnp.float32)
    oh = oh1 + oh2
    pre = jnp.dot(ltri_ref[...], oh, preferred_element_type=jnp.float32)
    base = carry[...] + pre
    r1 = jnp.sum(base * oh1, axis=-1, keepdims=True)
    r2 = jnp.sum(base * oh2, axis=-1, keepdims=True)
    r1_ref[...] = r1.astype(jnp.int32)
    r2_ref[...] = r2.astype(jnp.int32)
    carry[...] += jnp.sum(oh, axis=0, keepdims=True)
    cnt_ref[...] = carry[...]


def _router(x, Wg):
    return pl.pallas_call(
        _router_kernel,
        grid=(B // BT,),
        in_specs=[
            pl.BlockSpec((BT, IN_DIM), lambda i: (i, 0)),
            pl.BlockSpec((IN_DIM, E), lambda i: (0, 0)),
            pl.BlockSpec((BT, BT), lambda i: (0, 0)),
        ],
        out_specs=[
            pl.BlockSpec((BT, E), lambda i: (i, 0)),
            pl.BlockSpec((BT, 1), lambda i: (i, 0)),
            pl.BlockSpec((BT, 1), lambda i: (i, 0)),
            pl.BlockSpec((BT, 1), lambda i: (i, 0)),
            pl.BlockSpec((BT, 1), lambda i: (i, 0)),
            pl.BlockSpec((BT, 1), lambda i: (i, 0)),
            pl.BlockSpec((BT, 1), lambda i: (i, 0)),
            pl.BlockSpec((1, E), lambda i: (0, 0)),
        ],
        out_shape=[
            jax.ShapeDtypeStruct((B, E), jnp.float32),
            jax.ShapeDtypeStruct((B, 1), jnp.int32),
            jax.ShapeDtypeStruct((B, 1), jnp.int32),
            jax.ShapeDtypeStruct((B, 1), jnp.float32),
            jax.ShapeDtypeStruct((B, 1), jnp.float32),
            jax.ShapeDtypeStruct((B, 1), jnp.int32),
            jax.ShapeDtypeStruct((B, 1), jnp.int32),
            jax.ShapeDtypeStruct((1, E), jnp.float32),
        ],
        scratch_shapes=[pltpu.VMEM((1, E), jnp.float32)],
    )(x, Wg, jnp.tril(jnp.ones((BT, BT), jnp.float32), -1))


_SC_MESH = dict(core_axis_name="c", subcore_axis_name="s", num_cores=2,
                num_subcores=16)


def _worker_id():
    return lax.axis_index("s") * 2 + lax.axis_index("c")


@functools.cache
def _sc_dispatch_kernel():
    @functools.partial(
        pl.kernel,
        out_type=jax.ShapeDtypeStruct((S, IN_DIM), jnp.float32),
        mesh=plsc.VectorSubcoreMesh(**_SC_MESH),
        scratch_types=[
            pltpu.VMEM((DSUB, DCH), jnp.int32),
            pltpu.VMEM((DSUB, DCH), jnp.int32),
            pltpu.VMEM((DCH, IN_DIM), jnp.float32),
            pltpu.SemaphoreType.DMA,
            pltpu.SemaphoreType.DMA,
        ],
    )
    def body(x_hbm, tok_hbm, slot_hbm, xs_hbm, tok_v, slot_v, rows_v, sem_g, sem_s):
        wid = _worker_id()
        pltpu.sync_copy(tok_hbm.at[wid], tok_v)
        pltpu.sync_copy(slot_hbm.at[wid], slot_v)
        for c in range(DSUB):
            pltpu.async_copy(x_hbm.at[tok_v.at[c]], rows_v, sem_g).wait()
            pltpu.async_copy(rows_v, xs_hbm.at[slot_v.at[c]], sem_s).wait()

    return body


def _sc_dispatch(x, tok3, slot3):
    return _sc_dispatch_kernel()(x, tok3, slot3)


@functools.cache
def _sc_combine_kernel():
    @functools.partial(
        pl.kernel,
        out_type=(
            jax.ShapeDtypeStruct((B, OUT_DIM), jnp.float32),
            jax.ShapeDtypeStruct((B, OUT_DIM), jnp.float32),
        ),
        mesh=plsc.VectorSubcoreMesh(**_SC_MESH),
        scratch_types=[
            pltpu.VMEM((CSUB, CCH), jnp.int32),
            pltpu.VMEM((CSUB, CCH), jnp.int32),
            pltpu.VMEM((CCH, OUT_DIM), jnp.float32),
            pltpu.SemaphoreType.DMA,
        ],
    )
    def body(ys_hbm, p0_hbm, p1_hbm, y0_hbm, y1_hbm, p0_v, p1_v, rows_v, sem):
        wid = _worker_id()
        base = wid * BW
        pltpu.sync_copy(p0_hbm.at[wid], p0_v)
        pltpu.sync_copy(p1_hbm.at[wid], p1_v)
        for c in range(CSUB):
            pltpu.async_copy(ys_hbm.at[p0_v.at[c]], rows_v, sem).wait()
            pltpu.sync_copy(rows_v, y0_hbm.at[pl.ds(base + c * CCH, CCH)])
            pltpu.async_copy(ys_hbm.at[p1_v.at[c]], rows_v, sem).wait()
            pltpu.sync_copy(rows_v, y1_hbm.at[pl.ds(base + c * CCH, CCH)])

    return body


def _sc_combine(ys, pos0, pos1):
    return _sc_combine_kernel()(ys, pos0, pos1)


def _ffn_kernel(te_ref, act_ref, xs_ref, w1_ref, b1_ref, w2_ref,
                b2_ref, ys_ref):
    g = pl.program_id(0)

    @pl.when(act_ref[g] != 0)
    def _():
        xb = xs_ref[...].astype(jnp.bfloat16)
        h = jnp.dot(xb, w1_ref[0], preferred_element_type=jnp.float32) + b1_ref[0]
        h = jnp.maximum(h, 0.0).astype(jnp.bfloat16)
        y = jnp.dot(h, w2_ref[0], preferred_element_type=jnp.float32) + b2_ref[0]
        ys_ref[...] = y


def _ffn(te, act, xs, W1b, b1r, W2b, b2r):
    grid_spec = pltpu.PrefetchScalarGridSpec(
        num_scalar_prefetch=2,
        grid=(NT,),
        in_specs=[
            pl.BlockSpec((T, IN_DIM), lambda g, te, act: (g, 0)),
            pl.BlockSpec((1, IN_DIM, HID), lambda g, te, act: (te[g], 0, 0)),
            pl.BlockSpec((1, 1, HID), lambda g, te, act: (te[g], 0, 0)),
            pl.BlockSpec((1, HID, OUT_DIM), lambda g, te, act: (te[g], 0, 0)),
            pl.BlockSpec((1, 1, OUT_DIM), lambda g, te, act: (te[g], 0, 0)),
        ],
        out_specs=pl.BlockSpec((T, OUT_DIM), lambda g, te, act: (g, 0)),
    )
    return pl.pallas_call(
        _ffn_kernel,
        grid_spec=grid_spec,
        out_shape=jax.ShapeDtypeStruct((S, OUT_DIM), jnp.float32),
    )(te, act, xs, W1b, b1r, W2b, b2r)


def _mix_kernel(y0_ref, y1_ref, g1_ref, g2_ref, out_ref):
    out_ref[...] = y0_ref[...] * g1_ref[...] + y1_ref[...] * g2_ref[...]


def _mix(y0, y1, g1, g2):
    return pl.pallas_call(
        _mix_kernel,
        grid=(B // BT,),
        in_specs=[
            pl.BlockSpec((BT, OUT_DIM), lambda i: (i, 0)),
            pl.BlockSpec((BT, OUT_DIM), lambda i: (i, 0)),
            pl.BlockSpec((BT, 1), lambda i: (i, 0)),
            pl.BlockSpec((BT, 1), lambda i: (i, 0)),
        ],
        out_specs=pl.BlockSpec((BT, OUT_DIM), lambda i: (i, 0)),
        out_shape=jax.ShapeDtypeStruct((B, OUT_DIM), jnp.float32),
    )(y0, y1, g1, g2)


NCB = 8


def _cast_kernel(w1_ref, w2_ref, o1_ref, o2_ref):
    o1_ref[...] = w1_ref[...].astype(jnp.bfloat16)
    o2_ref[...] = w2_ref[...].astype(jnp.bfloat16)


def _cast(W1, W2):
    return pl.pallas_call(
        _cast_kernel,
        grid=(E, NCB),
        in_specs=[
            pl.BlockSpec((1, IN_DIM // NCB, HID), lambda e, i: (e, i, 0)),
            pl.BlockSpec((1, HID // NCB, OUT_DIM), lambda e, i: (e, i, 0)),
        ],
        out_specs=[
            pl.BlockSpec((1, IN_DIM // NCB, HID), lambda e, i: (e, i, 0)),
            pl.BlockSpec((1, HID // NCB, OUT_DIM), lambda e, i: (e, i, 0)),
        ],
        out_shape=[
            jax.ShapeDtypeStruct((E, IN_DIM, HID), jnp.bfloat16),
            jax.ShapeDtypeStruct((E, HID, OUT_DIM), jnp.bfloat16),
        ],
    )(W1, W2)


def kernel(x, Wg, W1, b1, W2, b2):
    probs, i1, i2, g1, g2, r1, r2, cntf = _router(x, Wg)

    counts = cntf.reshape(E).astype(jnp.int32)
    ntile = (counts + T - 1) // T
    pb = jnp.concatenate([jnp.zeros(1, ntile.dtype), jnp.cumsum(ntile)[:-1]])
    er = jnp.arange(E, dtype=jnp.int32)
    pb1 = jnp.sum((i1 == er[None, :]) * pb[None, :], axis=1, keepdims=True)
    pb2 = jnp.sum((i2 == er[None, :]) * pb[None, :], axis=1, keepdims=True)
    slot1 = T * pb1 + r1
    slot2 = T * pb2 + r2
    slot = jnp.concatenate([slot1, slot2], axis=1).reshape(A)
    total_tiles = pb[E - 1] + ntile[E - 1]
    tile_expert = (
        jnp.sum(jnp.arange(NT)[:, None] >= pb[None, :], axis=1) - 1
    ).astype(jnp.int32)
    act = (jnp.arange(NT) < total_tiles).astype(jnp.int32)

    pos0 = slot1.reshape(NW, CSUB, CCH)
    pos1 = slot2.reshape(NW, CSUB, CCH)
    slot3 = slot.reshape(NW, DSUB, DCH)
    tok3 = (jnp.arange(A, dtype=jnp.int32) // TOP_K).reshape(NW, DSUB, DCH)
    xs = _sc_dispatch(x, tok3, slot3)

    W1b, W2b = _cast(W1, W2)
    ys = _ffn(
        tile_expert,
        act,
        xs,
        W1b,
        b1.reshape(E, 1, HID),
        W2b,
        b2.reshape(E, 1, OUT_DIM),
    )

    y0, y1 = _sc_combine(ys, pos0, pos1)
    out = _mix(y0, y1, g1, g2)
    return (out, probs)
